# baseline (speedup 1.0000x reference)
import functools

import jax
import jax.numpy as jnp
from jax import lax
from jax.experimental import pallas as pl
from jax.experimental.pallas import tpu as pltpu
from jax.experimental.pallas import tpu_sc as plsc

NBUF = 2
C = 2


def _make_sc_kernel(B, L, H):
    info = plsc.get_sparse_core_info()
    NC, NS = info.num_cores, info.num_subcores
    NW = NC * NS
    rows_per_w = B // NW
    chunks = rows_per_w // C
    mesh = plsc.VectorSubcoreMesh(core_axis_name="c", subcore_axis_name="s")

    @functools.partial(
        pl.kernel,
        mesh=mesh,
        out_type=jax.ShapeDtypeStruct((B, L, H), jnp.float32),
        scratch_types=[
            pltpu.VMEM((NBUF, C, L, H), jnp.float32),
        ]
        + [pltpu.SemaphoreType.DMA] * (2 * NBUF),
    )
    def k(x_hbm, pos_hbm, out_hbm, shared, *sems):
        in_sem = sems[0:NBUF]
        out_sem = sems[NBUF:2 * NBUF]
        cid = lax.axis_index("c")
        sid = lax.axis_index("s")
        wid = sid * NC + cid
        base = wid * rows_per_w

        def slot(p):
            return p

        def start_in(ci, p):
            pltpu.async_copy(x_hbm.at[pl.ds(base + ci * C, C)],
                             shared.at[slot(p)], in_sem[p])

        def wait_in(ci, p):
            pltpu.make_async_copy(x_hbm.at[pl.ds(base + ci * C, C)],
                                  shared.at[slot(p)], in_sem[p]).wait()

        def start_out(ci, p):
            pltpu.async_copy(shared.at[slot(p)],
                             out_hbm.at[pl.ds(base + ci * C, C)], out_sem[p])

        def wait_out(ci, p):
            pltpu.make_async_copy(shared.at[slot(p)],
                                  out_hbm.at[pl.ds(base + ci * C, C)],
                                  out_sem[p]).wait()

        start_in(0, 0)
        start_in(1, 1)
        for p in range(NBUF):
            wait_in(p, p)
            start_out(p, p)

        def body(t, carry):
            g = t * NBUF
            for p in range(NBUF):
                ci = g + p
                wait_out(ci - 2, p)
                start_in(ci, p)
                wait_in(ci, p)
                start_out(ci, p)
            return carry

        lax.fori_loop(1, chunks // NBUF, body, 0)

        for p in range(NBUF):
            wait_out(chunks - NBUF + p, p)

    return k


def kernel(x, pos_table):
    B, L, H = x.shape
    k = _make_sc_kernel(B, L, H)
    return k(x, pos_table[:L])


# speedup vs baseline: 1.0773x; 1.0773x over previous
import functools

import jax
import jax.numpy as jnp
from jax import lax
from jax.experimental import pallas as pl
from jax.experimental.pallas import tpu as pltpu
from jax.experimental.pallas import tpu_sc as plsc

IDX_SPLIT = 128
NBUF = 4


def _make_sc_kernel(B, L, H):
    info = plsc.get_sparse_core_info()
    NC, NS = info.num_cores, info.num_subcores
    NW = NC * NS
    rows_per_w = B // NW
    mesh = plsc.VectorSubcoreMesh(core_axis_name="c", subcore_axis_name="s")
    L1 = L - IDX_SPLIT

    @functools.partial(
        pl.kernel,
        mesh=mesh,
        out_type=jax.ShapeDtypeStruct((B, L, H), jnp.float32),
        scratch_types=[
            pltpu.VMEM((L, H), jnp.float32),
            pltpu.VMEM((NBUF, IDX_SPLIT), jnp.int32),
            pltpu.VMEM((NBUF, L1), jnp.int32),
            pltpu.VMEM_SHARED((NS * NBUF * L, H), jnp.float32),
        ]
        + [pltpu.SemaphoreType.DMA] * (3 * NBUF),
    )
    def k(x_hbm, pos_hbm, idx0_hbm, idx1_hbm, out_hbm, pos_v, idx0_v, idx1_v,
          shared, *sems):
        in_sem = sems[0:NBUF]
        add_sem = sems[NBUF:2 * NBUF]
        out_sem = sems[2 * NBUF:3 * NBUF]
        cid = lax.axis_index("c")
        sid = lax.axis_index("s")
        wid = sid * NC + cid
        base = wid * rows_per_w
        pltpu.sync_copy(pos_hbm, pos_v)
        pltpu.sync_copy(idx0_hbm.at[sid], idx0_v)
        pltpu.sync_copy(idx1_hbm.at[sid], idx1_v)

        def slot(p):
            return pl.ds((sid * NBUF + p) * L, L)

        def start_in(row, p):
            pltpu.async_copy(x_hbm.at[row], shared.at[slot(p)], in_sem[p])

        def wait_in(row, p):
            pltpu.make_async_copy(x_hbm.at[row], shared.at[slot(p)],
                                  in_sem[p]).wait()

        def start_adds(p):
            pltpu.async_copy(pos_v.at[pl.ds(0, IDX_SPLIT)],
                             shared.at[idx0_v.at[p]], add_sem[p], add=True)
            pltpu.async_copy(pos_v.at[pl.ds(IDX_SPLIT, L1)],
                             shared.at[idx1_v.at[p]], add_sem[p], add=True)

        def wait_adds(p):
            pltpu.make_async_copy(pos_v.at[pl.ds(0, IDX_SPLIT)],
                                  shared.at[idx0_v.at[p]], add_sem[p]).wait()
            pltpu.make_async_copy(pos_v.at[pl.ds(IDX_SPLIT, L1)],
                                  shared.at[idx1_v.at[p]], add_sem[p]).wait()

        def start_out(row, p):
            pltpu.async_copy(shared.at[slot(p)], out_hbm.at[row], out_sem[p])

        def wait_out(row, p):
            pltpu.make_async_copy(shared.at[slot(p)], out_hbm.at[row],
                                  out_sem[p]).wait()

        start_in(base + 0, 0)
        start_in(base + 1, 1)
        for p in range(NBUF):
            i = p
            wait_in(base + i, p)
            start_adds(p)
            if p >= 1:
                wait_adds(p - 1)
                start_out(base + i - 1, p - 1)
            if p < 2:
                start_in(base + i + 2, (i + 2) % NBUF)
            else:
                q = (p + 2) % NBUF
                wait_out(base + q, q)
                start_in(base + i + 2, q)

        def body(t, carry):
            g = t * NBUF
            for p in range(NBUF):
                i = g + p
                row = base + i
                pm = (p - 1) % NBUF
                wait_in(row, p)
                start_adds(p)
                wait_adds(pm)
                start_out(row - 1, pm)
                q = (p + 2) % NBUF
                j = i + 2

                @pl.when(j < rows_per_w)
                def _():
                    wait_out(row - 2, q)
                    start_in(base + j, q)

            return carry

        lax.fori_loop(1, rows_per_w // NBUF, body, 0)

        last = rows_per_w - 1
        wait_adds(NBUF - 1)
        start_out(base + last, NBUF - 1)
        for p in range(NBUF):
            wait_out(base + rows_per_w - NBUF + p, p)

    return k, NS


def kernel(x, pos_table):
    B, L, H = x.shape
    k, ns = _make_sc_kernel(B, L, H)
    slot_base = (jnp.arange(ns, dtype=jnp.int32)[:, None] * NBUF
                 + jnp.arange(NBUF, dtype=jnp.int32)[None, :]) * L
    idx0 = slot_base[:, :, None] + jnp.arange(IDX_SPLIT, dtype=jnp.int32)
    idx1 = slot_base[:, :, None] + jnp.arange(IDX_SPLIT, L, dtype=jnp.int32)
    return k(x, pos_table[:L], idx0, idx1)
